# Initial kernel scaffold; baseline (speedup 1.0000x reference)
import jax
import jax.numpy as jnp
from jax.experimental import pallas as pl


def kernel(node_feats, coords, W_tp_a, W_tp_b, W_lin_s, W_lin_v):
    raise NotImplementedError("write your pallas kernel here")


# speedup vs baseline: 13.9275x; 13.9275x over previous
import functools
import math

import jax
import jax.numpy as jnp
from jax.experimental import pallas as pl
from jax.experimental.pallas import tpu as pltpu

B, N, K = 8, 2048, 8
MUL0, MUL1 = 128, 64
D = MUL0 + 3 * MUL1
RB = 256
BIG = 1e30


def _lrb_kernel(feats_ref, vp_ref, coords_ref, wav_ref, wcb_ref, out_ref):
    s = feats_ref[0]
    vp = vp_ref[0]
    c = coords_ref[0]

    t = jnp.dot(s, wav_ref[:, :], preferred_element_type=jnp.float32, precision=jax.lax.Precision.HIGHEST)
    c0 = c[:, 0:1]
    c1 = c[:, 1:2]
    c2 = c[:, 2:3]
    v0 = vp[:, 0:64]
    v1 = vp[:, 64:128]
    v2 = vp[:, 128:192]
    p = v0 * c0 + v1 * c1 + v2 * c2
    F = jnp.concatenate(
        [t, t * c0, t * c1, t * c2, p, vp], axis=1)

    sq = jnp.sum(c * c, axis=1, keepdims=True)

    for blk in range(N // RB):
        r0 = blk * RB
        cb = c[r0:r0 + RB]
        sqb = sq[r0:r0 + RB]
        g = jax.lax.dot_general(
            cb, c, (((1,), (1,)), ((), ())),
            preferred_element_type=jnp.float32)
        d2 = sqb + sq[:, 0][None, :] - 2.0 * g
        rows = r0 + jax.lax.broadcasted_iota(jnp.int32, (RB, N), 0)
        cols = jax.lax.broadcasted_iota(jnp.int32, (RB, N), 1)
        d2 = jnp.where(rows == cols, BIG, d2)
        work = d2
        rowmin = jnp.min(work, axis=1, keepdims=True)
        for _ in range(K - 1):
            work = jnp.where(work <= rowmin, BIG, work)
            rowmin = jnp.min(work, axis=1, keepdims=True)
        adj = (d2 <= rowmin).astype(jnp.float32)
        agg = jnp.dot(adj, F, preferred_element_type=jnp.float32, precision=jax.lax.Precision.HIGHEST)

        cb0 = cb[:, 0:1]
        cb1 = cb[:, 1:2]
        cb2 = cb[:, 2:3]
        aggT = agg[:, 0:64]
        dots = (agg[:, 256:320]
                - agg[:, 320:384] * cb0
                - agg[:, 384:448] * cb1
                - agg[:, 448:512] * cb2)
        out_s = jnp.dot(dots, wcb_ref[:, :],
                        preferred_element_type=jnp.float32, precision=jax.lax.Precision.HIGHEST)
        out_ref[0, r0:r0 + RB, 0:128] = out_s
        out_ref[0, r0:r0 + RB, 128:192] = agg[:, 64:128] - aggT * cb0
        out_ref[0, r0:r0 + RB, 192:256] = agg[:, 128:192] - aggT * cb1
        out_ref[0, r0:r0 + RB, 256:320] = agg[:, 192:256] - aggT * cb2


@jax.jit
def kernel(node_feats, coords, W_tp_a, W_tp_b, W_lin_s, W_lin_v):
    c_a = 1.0 / math.sqrt(MUL0)
    c_b = 1.0 / math.sqrt(MUL1 * 3)
    wav = (c_a / math.sqrt(MUL1)) * (W_tp_a @ W_lin_v)
    wcb = (c_b / math.sqrt(MUL0)) * (W_tp_b @ W_lin_s)

    s = node_feats[:, :, :MUL0]
    vp = (node_feats[:, :, MUL0:]
          .reshape(B, N, MUL1, 3)
          .transpose(0, 1, 3, 2)
          .reshape(B, N, 3 * MUL1))

    out = pl.pallas_call(
        _lrb_kernel,
        grid=(B,),
        in_specs=[
            pl.BlockSpec((1, N, MUL0), lambda b: (b, 0, 0)),
            pl.BlockSpec((1, N, 3 * MUL1), lambda b: (b, 0, 0)),
            pl.BlockSpec((1, N, 3), lambda b: (b, 0, 0)),
            pl.BlockSpec((MUL0, MUL1), lambda b: (0, 0)),
            pl.BlockSpec((MUL1, MUL0), lambda b: (0, 0)),
        ],
        out_specs=pl.BlockSpec((1, N, D), lambda b: (b, 0, 0)),
        out_shape=jax.ShapeDtypeStruct((B, N, D), jnp.float32),
    )(s, vp, coords, wav, wcb)

    out_s = out[:, :, :MUL0]
    out_v = (out[:, :, MUL0:]
             .reshape(B, N, 3, MUL1)
             .transpose(0, 1, 3, 2)
             .reshape(B, N, 3 * MUL1))
    return jnp.concatenate([out_s, out_v], axis=-1)
